# baseline (speedup 1.0000x reference)
import functools

import jax
import jax.numpy as jnp
from jax import lax
from jax.experimental import pallas as pl
from jax.experimental.pallas import tpu as pltpu
from jax.experimental.pallas import tpu_sc as plsc

N_IN = 71945
NPAD = 73728
RPT = NPAD // 16
RPW = NPAD // 32
SBR = 288
E_PAD = 1048576
NCH = 256
BCH = 64
H = 64
HC = 16
BLK = 1024

_mesh = plsc.VectorSubcoreMesh(core_axis_name="c", subcore_axis_name="s")
_sc_params = pltpu.CompilerParams(use_tc_tiling_on_sc=False)


@functools.partial(
    pl.kernel,
    out_type=jax.ShapeDtypeStruct((2, NPAD), jnp.float32),
    mesh=_mesh,
    compiler_params=_sc_params,
    scratch_types=[
        pltpu.VMEM((NCH, 128), jnp.int32),
        pltpu.VMEM((128,), jnp.float32),
        pltpu.VMEM_SHARED((NPAD,), jnp.float32),
    ],
)
def _deg_kernel(dstp, ones_h, zdeg, deg_part, dst_t, ones_t, deg_s):
    c = lax.axis_index("c")
    s = lax.axis_index("s")
    wid = c * 16 + s
    pltpu.sync_copy(dstp.at[wid], dst_t)
    pltpu.sync_copy(ones_h, ones_t)
    pltpu.sync_copy(zdeg, deg_s.at[pl.ds(s * RPT, RPT)])
    plsc.subcore_barrier()

    @pl.loop(0, NCH)
    def _(j):
        pltpu.sync_copy(ones_t, deg_s.at[dst_t.at[j]], add=True)

    plsc.subcore_barrier()
    pltpu.sync_copy(deg_s.at[pl.ds(s * RPT, RPT)],
                    deg_part.at[c, pl.ds(s * RPT, RPT)])


@functools.partial(
    pl.kernel,
    out_type=[jax.ShapeDtypeStruct((NPAD, HC), jnp.float32)
              for _ in range(8)],
    mesh=_mesh,
    compiler_params=_sc_params,
    scratch_types=[
        pltpu.VMEM((BCH, 128), jnp.int32),
        pltpu.VMEM((BCH, 128), jnp.int32),
        pltpu.VMEM((128, HC), jnp.float32),
        pltpu.VMEM_SHARED((NPAD, HC), jnp.float32),
    ],
)
def _prop_kernel(w0, w1, w2, w3, srcp, dstp, zrows,
                 sa0, sa1, sa2, sa3, sb0, sb1, sb2, sb3,
                 src_t, dst_t, rows, acc):
    c = lax.axis_index("c")
    s = lax.axis_index("s")
    wid = c * 16 + s
    for w, spa, spb in ((w0, sa0, sb0), (w1, sa1, sb1),
                        (w2, sa2, sb2), (w3, sa3, sb3)):
        pltpu.sync_copy(zrows, acc.at[pl.ds(s * RPT, RPT), :])
        plsc.subcore_barrier()

        @pl.loop(0, NCH // BCH)
        def _(b):
            pltpu.sync_copy(srcp.at[wid, pl.ds(b * BCH, BCH)], src_t)
            pltpu.sync_copy(dstp.at[wid, pl.ds(b * BCH, BCH)], dst_t)

            @pl.loop(0, BCH)
            def _(j):
                pltpu.sync_copy(w.at[src_t.at[j]], rows)
                pltpu.sync_copy(rows, acc.at[dst_t.at[j]], add=True)

        plsc.subcore_barrier()

        @pl.when(c == 0)
        def _():
            pltpu.sync_copy(acc.at[pl.ds(s * RPT, RPT), :],
                            spa.at[pl.ds(s * RPT, RPT), :])

        @pl.when(c == 1)
        def _():
            pltpu.sync_copy(acc.at[pl.ds(s * RPT, RPT), :],
                            spb.at[pl.ds(s * RPT, RPT), :])


@functools.partial(
    pl.kernel,
    out_type=[jax.ShapeDtypeStruct((NPAD, HC), jnp.float32)
              for _ in range(8)],
    mesh=_mesh,
    compiler_params=_sc_params,
    scratch_types=[
        pltpu.VMEM((SBR, HC), jnp.float32),
        pltpu.VMEM((SBR, HC), jnp.float32),
        pltpu.VMEM((SBR, HC), jnp.float32),
        pltpu.VMEM((SBR, HC), jnp.float32),
        pltpu.VMEM((SBR, HC), jnp.float32),
        pltpu.VMEM((SBR, HC), jnp.float32),
    ],
)
def _scomb_kernel(sa0, sa1, sa2, sa3, sb0, sb1, sb2, sb3, db, d2,
                  wo0, wo1, wo2, wo3, xo0, xo1, xo2, xo3,
                  sat, sbt, dbt, d2t, wt, xt):
    c = lax.axis_index("c")
    s = lax.axis_index("s")
    wid = c * 16 + s
    base = wid * RPW

    @pl.loop(0, RPW // SBR)
    def _(b):
        off = base + b * SBR
        pltpu.sync_copy(db.at[pl.ds(off, SBR), :], dbt)
        pltpu.sync_copy(d2.at[pl.ds(off, SBR), :], d2t)
        for sa, sb, wo, xo in ((sa0, sb0, wo0, xo0), (sa1, sb1, wo1, xo1),
                               (sa2, sb2, wo2, xo2), (sa3, sb3, wo3, xo3)):
            pltpu.sync_copy(sa.at[pl.ds(off, SBR), :], sat)
            pltpu.sync_copy(sb.at[pl.ds(off, SBR), :], sbt)

            @pl.loop(0, SBR)
            def _(i):
                t = sat[i, :] + sbt[i, :]
                wt[i, :] = t * d2t[i, :]
                xt[i, :] = t * dbt[i, :]

            pltpu.sync_copy(wt, wo.at[pl.ds(off, SBR), :])
            pltpu.sync_copy(xt, xo.at[pl.ds(off, SBR), :])


@functools.partial(
    pl.kernel,
    out_type=[jax.ShapeDtypeStruct((NPAD, HC), jnp.float32)
              for _ in range(4)],
    mesh=_mesh,
    compiler_params=_sc_params,
    scratch_types=[
        pltpu.VMEM((SBR, HC), jnp.float32),
        pltpu.VMEM((SBR, HC), jnp.float32),
        pltpu.VMEM((SBR, HC), jnp.float32),
        pltpu.VMEM((SBR, HC), jnp.float32),
        pltpu.VMEM((SBR, HC), jnp.float32),
    ],
)
def _sfinal_kernel(e0, e1, e2, e3, x10, x11, x12, x13,
                   x20, x21, x22, x23, x30, x31, x32, x33,
                   o0, o1, o2, o3,
                   et, x1t, x2t, x3t, ot):
    c = lax.axis_index("c")
    s = lax.axis_index("s")
    wid = c * 16 + s
    base = wid * RPW

    @pl.loop(0, RPW // SBR)
    def _(b):
        off = base + b * SBR
        for e, x1, x2, x3, o in ((e0, x10, x20, x30, o0),
                                 (e1, x11, x21, x31, o1),
                                 (e2, x12, x22, x32, o2),
                                 (e3, x13, x23, x33, o3)):
            pltpu.sync_copy(e.at[pl.ds(off, SBR), :], et)
            pltpu.sync_copy(x1.at[pl.ds(off, SBR), :], x1t)
            pltpu.sync_copy(x2.at[pl.ds(off, SBR), :], x2t)
            pltpu.sync_copy(x3.at[pl.ds(off, SBR), :], x3t)

            @pl.loop(0, SBR)
            def _(i):
                ot[i, :] = (et[i, :] + x1t[i, :] + x2t[i, :]
                            + x3t[i, :]) * 0.25

            pltpu.sync_copy(ot, o.at[pl.ds(off, SBR), :])


def _dis_body(dp_ref, dis_ref):
    dp = dp_ref[...]
    deg = dp[:, 0:1] + dp[:, 1:2]
    dis_ref[...] = jnp.where(deg > 0.0, lax.rsqrt(jnp.maximum(deg, 1.0)), 0.0)


def _prep_body(emb_ref, dis_ref, w0, w1, w2, w3, db, d2):
    dis = dis_ref[...]
    x = emb_ref[...] * dis
    for r, wr in enumerate((w0, w1, w2, w3)):
        wr[...] = x[:, r * HC:(r + 1) * HC]
    db[...] = jnp.broadcast_to(dis, (BLK, HC))
    d2[...] = jnp.broadcast_to(dis * dis, (BLK, HC))


_spec_dis = pl.BlockSpec((BLK, 1), lambda i: (i, 0))
_spec_full = pl.BlockSpec((BLK, H), lambda i: (i, 0))
_spec_w = pl.BlockSpec((BLK, HC), lambda i: (i, 0))

_dis_call = pl.pallas_call(
    _dis_body,
    out_shape=jax.ShapeDtypeStruct((NPAD, 1), jnp.float32),
    grid=(NPAD // 4608,),
    in_specs=[pl.BlockSpec((4608, 2), lambda i: (i, 0))],
    out_specs=pl.BlockSpec((4608, 1), lambda i: (i, 0)),
)

_prep_call = pl.pallas_call(
    _prep_body,
    out_shape=[jax.ShapeDtypeStruct((NPAD, HC), jnp.float32)] * 6,
    grid=(NPAD // BLK,),
    in_specs=[_spec_full, _spec_dis],
    out_specs=[_spec_w] * 6,
)


def kernel(emb, edge_index):
    emb_pad = jnp.zeros((NPAD, H), jnp.float32).at[:N_IN].set(emb)
    src = edge_index[0]
    dst = edge_index[1]
    pad_e = E_PAD - src.shape[0]
    srcp = jnp.concatenate(
        [src, jnp.zeros((pad_e,), jnp.int32)]).reshape(32, NCH, 128)
    dstp = jnp.concatenate(
        [dst, jnp.full((pad_e,), NPAD - 1, jnp.int32)]).reshape(32, NCH, 128)
    zrows = jnp.zeros((RPT, HC), jnp.float32)
    zdeg = jnp.zeros((RPT,), jnp.float32)
    ones_h = jnp.ones((128,), jnp.float32)

    deg_part = _deg_kernel(dstp, ones_h, zdeg)
    dis = _dis_call(deg_part.T)
    res = _prep_call(emb_pad, dis)
    w, db, d2 = res[:4], res[4], res[5]

    xs = []
    for _ in range(3):
        sps = _prop_kernel(*w, srcp, dstp, zrows)
        res = _scomb_kernel(*sps, db, d2)
        w, x = res[:4], res[4:]
        xs.append(x)

    e_chunks = [lax.slice_in_dim(emb_pad, r * HC, (r + 1) * HC, axis=1)
                for r in range(4)]
    outs = _sfinal_kernel(*e_chunks, *xs[0], *xs[1], *xs[2])
    return jnp.concatenate(outs, axis=1)[:N_IN]


# speedup vs baseline: 1.1055x; 1.1055x over previous
import functools

import jax
import jax.numpy as jnp
from jax import lax
from jax.experimental import pallas as pl
from jax.experimental.pallas import tpu as pltpu
from jax.experimental.pallas import tpu_sc as plsc

N_IN = 71945
NPAD = 73728
RPT = NPAD // 16
RPW = NPAD // 32
SBR = 288
E_PAD = 1048576
NCH = 256
BCH = 64
H = 64
HC = 16
BLK = 1024

_mesh = plsc.VectorSubcoreMesh(core_axis_name="c", subcore_axis_name="s")
_sc_params = pltpu.CompilerParams(use_tc_tiling_on_sc=False)


@functools.partial(
    pl.kernel,
    out_type=jax.ShapeDtypeStruct((2, NPAD), jnp.float32),
    mesh=_mesh,
    compiler_params=_sc_params,
    scratch_types=[
        pltpu.VMEM((NCH, 128), jnp.int32),
        pltpu.VMEM((128,), jnp.float32),
        pltpu.VMEM_SHARED((NPAD,), jnp.float32),
        pltpu.SemaphoreType.DMA,
        pltpu.SemaphoreType.DMA,
        pltpu.SemaphoreType.DMA,
        pltpu.SemaphoreType.DMA,
    ],
)
def _deg_kernel(dstp, ones_h, zdeg, deg_part, dst_t, ones_t, deg_s,
                sm0, sm1, sm2, sm3):
    c = lax.axis_index("c")
    s = lax.axis_index("s")
    wid = c * 16 + s
    sems = (sm0, sm1, sm2, sm3)
    pltpu.sync_copy(dstp.at[wid], dst_t)
    pltpu.sync_copy(ones_h, ones_t)
    pltpu.sync_copy(zdeg, deg_s.at[pl.ds(s * RPT, RPT)])
    plsc.subcore_barrier()

    @pl.loop(0, NCH // 4)
    def _(j4):
        ds_ = [pltpu.async_copy(ones_t, deg_s.at[dst_t.at[j4 * 4 + k]],
                                sems[k], add=True) for k in range(4)]
        for d in ds_:
            d.wait()

    plsc.subcore_barrier()
    pltpu.sync_copy(deg_s.at[pl.ds(s * RPT, RPT)],
                    deg_part.at[c, pl.ds(s * RPT, RPT)])


@functools.partial(
    pl.kernel,
    out_type=[jax.ShapeDtypeStruct((NPAD, HC), jnp.float32)
              for _ in range(8)],
    mesh=_mesh,
    compiler_params=_sc_params,
    scratch_types=[
        pltpu.VMEM((BCH, 128), jnp.int32),
        pltpu.VMEM((BCH, 128), jnp.int32),
        pltpu.VMEM((8, 128, HC), jnp.float32),
        pltpu.VMEM_SHARED((NPAD, HC), jnp.float32),
    ] + [pltpu.SemaphoreType.DMA] * 16,
)
def _prop_kernel(w0, w1, w2, w3, srcp, dstp, zrows,
                 sa0, sa1, sa2, sa3, sb0, sb1, sb2, sb3,
                 src_t, dst_t, rows, acc, *sems):
    c = lax.axis_index("c")
    s = lax.axis_index("s")
    wid = c * 16 + s
    gsems = sems[:8]
    ssems = sems[8:]
    for w, spa, spb in ((w0, sa0, sb0), (w1, sa1, sb1),
                        (w2, sa2, sb2), (w3, sa3, sb3)):
        pltpu.sync_copy(zrows, acc.at[pl.ds(s * RPT, RPT), :])
        plsc.subcore_barrier()

        for b in range(NCH // BCH):
            pltpu.sync_copy(srcp.at[wid, pl.ds(b * BCH, BCH)], src_t)
            pltpu.sync_copy(dstp.at[wid, pl.ds(b * BCH, BCH)], dst_t)

            @pl.loop(0, BCH // 8)
            def _(j8):
                gds = [pltpu.async_copy(w.at[src_t.at[j8 * 8 + k]],
                                        rows.at[k], gsems[k])
                       for k in range(8)]
                sds = []
                for k in range(8):
                    gds[k].wait()
                    sds.append(pltpu.async_copy(
                        rows.at[k], acc.at[dst_t.at[j8 * 8 + k]],
                        ssems[k], add=True))
                for d in sds:
                    d.wait()

        plsc.subcore_barrier()

        @pl.when(c == 0)
        def _():
            pltpu.sync_copy(acc.at[pl.ds(s * RPT, RPT), :],
                            spa.at[pl.ds(s * RPT, RPT), :])

        @pl.when(c == 1)
        def _():
            pltpu.sync_copy(acc.at[pl.ds(s * RPT, RPT), :],
                            spb.at[pl.ds(s * RPT, RPT), :])


@functools.partial(
    pl.kernel,
    out_type=[jax.ShapeDtypeStruct((NPAD, HC), jnp.float32)
              for _ in range(8)],
    mesh=_mesh,
    compiler_params=_sc_params,
    scratch_types=[
        pltpu.VMEM((SBR, HC), jnp.float32),
        pltpu.VMEM((SBR, HC), jnp.float32),
        pltpu.VMEM((SBR, HC), jnp.float32),
        pltpu.VMEM((SBR, HC), jnp.float32),
        pltpu.VMEM((SBR, HC), jnp.float32),
        pltpu.VMEM((SBR, HC), jnp.float32),
    ],
)
def _scomb_kernel(sa0, sa1, sa2, sa3, sb0, sb1, sb2, sb3, db, d2,
                  wo0, wo1, wo2, wo3, xo0, xo1, xo2, xo3,
                  sat, sbt, dbt, d2t, wt, xt):
    c = lax.axis_index("c")
    s = lax.axis_index("s")
    wid = c * 16 + s
    base = wid * RPW

    @pl.loop(0, RPW // SBR)
    def _(b):
        off = base + b * SBR
        pltpu.sync_copy(db.at[pl.ds(off, SBR), :], dbt)
        pltpu.sync_copy(d2.at[pl.ds(off, SBR), :], d2t)
        for sa, sb, wo, xo in ((sa0, sb0, wo0, xo0), (sa1, sb1, wo1, xo1),
                               (sa2, sb2, wo2, xo2), (sa3, sb3, wo3, xo3)):
            pltpu.sync_copy(sa.at[pl.ds(off, SBR), :], sat)
            pltpu.sync_copy(sb.at[pl.ds(off, SBR), :], sbt)

            @pl.loop(0, SBR)
            def _(i):
                t = sat[i, :] + sbt[i, :]
                wt[i, :] = t * d2t[i, :]
                xt[i, :] = t * dbt[i, :]

            pltpu.sync_copy(wt, wo.at[pl.ds(off, SBR), :])
            pltpu.sync_copy(xt, xo.at[pl.ds(off, SBR), :])


@functools.partial(
    pl.kernel,
    out_type=[jax.ShapeDtypeStruct((NPAD, HC), jnp.float32)
              for _ in range(4)],
    mesh=_mesh,
    compiler_params=_sc_params,
    scratch_types=[
        pltpu.VMEM((SBR, HC), jnp.float32),
        pltpu.VMEM((SBR, HC), jnp.float32),
        pltpu.VMEM((SBR, HC), jnp.float32),
        pltpu.VMEM((SBR, HC), jnp.float32),
        pltpu.VMEM((SBR, HC), jnp.float32),
    ],
)
def _sfinal_kernel(e0, e1, e2, e3, x10, x11, x12, x13,
                   x20, x21, x22, x23, x30, x31, x32, x33,
                   o0, o1, o2, o3,
                   et, x1t, x2t, x3t, ot):
    c = lax.axis_index("c")
    s = lax.axis_index("s")
    wid = c * 16 + s
    base = wid * RPW

    @pl.loop(0, RPW // SBR)
    def _(b):
        off = base + b * SBR
        for e, x1, x2, x3, o in ((e0, x10, x20, x30, o0),
                                 (e1, x11, x21, x31, o1),
                                 (e2, x12, x22, x32, o2),
                                 (e3, x13, x23, x33, o3)):
            pltpu.sync_copy(e.at[pl.ds(off, SBR), :], et)
            pltpu.sync_copy(x1.at[pl.ds(off, SBR), :], x1t)
            pltpu.sync_copy(x2.at[pl.ds(off, SBR), :], x2t)
            pltpu.sync_copy(x3.at[pl.ds(off, SBR), :], x3t)

            @pl.loop(0, SBR)
            def _(i):
                ot[i, :] = (et[i, :] + x1t[i, :] + x2t[i, :]
                            + x3t[i, :]) * 0.25

            pltpu.sync_copy(ot, o.at[pl.ds(off, SBR), :])


def _dis_body(dp_ref, dis_ref):
    dp = dp_ref[...]
    deg = dp[:, 0:1] + dp[:, 1:2]
    dis_ref[...] = jnp.where(deg > 0.0, lax.rsqrt(jnp.maximum(deg, 1.0)), 0.0)


def _prep_body(emb_ref, dis_ref, w0, w1, w2, w3, db, d2):
    dis = dis_ref[...]
    x = emb_ref[...] * dis
    for r, wr in enumerate((w0, w1, w2, w3)):
        wr[...] = x[:, r * HC:(r + 1) * HC]
    db[...] = jnp.broadcast_to(dis, (BLK, HC))
    d2[...] = jnp.broadcast_to(dis * dis, (BLK, HC))


_spec_dis = pl.BlockSpec((BLK, 1), lambda i: (i, 0))
_spec_full = pl.BlockSpec((BLK, H), lambda i: (i, 0))
_spec_w = pl.BlockSpec((BLK, HC), lambda i: (i, 0))

_dis_call = pl.pallas_call(
    _dis_body,
    out_shape=jax.ShapeDtypeStruct((NPAD, 1), jnp.float32),
    grid=(NPAD // 4608,),
    in_specs=[pl.BlockSpec((4608, 2), lambda i: (i, 0))],
    out_specs=pl.BlockSpec((4608, 1), lambda i: (i, 0)),
)

_prep_call = pl.pallas_call(
    _prep_body,
    out_shape=[jax.ShapeDtypeStruct((NPAD, HC), jnp.float32)] * 6,
    grid=(NPAD // BLK,),
    in_specs=[_spec_full, _spec_dis],
    out_specs=[_spec_w] * 6,
)


def kernel(emb, edge_index):
    emb_pad = jnp.zeros((NPAD, H), jnp.float32).at[:N_IN].set(emb)
    src = edge_index[0]
    dst = edge_index[1]
    pad_e = E_PAD - src.shape[0]
    srcp = jnp.concatenate(
        [src, jnp.zeros((pad_e,), jnp.int32)]).reshape(32, NCH, 128)
    dstp = jnp.concatenate(
        [dst, jnp.full((pad_e,), NPAD - 1, jnp.int32)]).reshape(32, NCH, 128)
    zrows = jnp.zeros((RPT, HC), jnp.float32)
    zdeg = jnp.zeros((RPT,), jnp.float32)
    ones_h = jnp.ones((128,), jnp.float32)

    deg_part = _deg_kernel(dstp, ones_h, zdeg)
    dis = _dis_call(deg_part.T)
    res = _prep_call(emb_pad, dis)
    w, db, d2 = res[:4], res[4], res[5]

    xs = []
    for _ in range(3):
        sps = _prop_kernel(*w, srcp, dstp, zrows)
        res = _scomb_kernel(*sps, db, d2)
        w, x = res[:4], res[4:]
        xs.append(x)

    e_chunks = [lax.slice_in_dim(emb_pad, r * HC, (r + 1) * HC, axis=1)
                for r in range(4)]
    outs = _sfinal_kernel(*e_chunks, *xs[0], *xs[1], *xs[2])
    return jnp.concatenate(outs, axis=1)[:N_IN]
